# baseline (speedup 1.0000x reference)
import functools

import jax
import jax.numpy as jnp
import numpy as np
from jax import lax
from jax.experimental import pallas as pl
from jax.experimental.pallas import tpu as pltpu
from jax.experimental.pallas import tpu_sc as plsc

B = 4096
K = 128
NW = 32
ROWS_W = B // NW
LANES = 16

V_PAD = 100352
ZCH = V_PAD // 16

AUG = 136
TILE = 1024
T = B // TILE
_PAIRS = np.array(
    [(i, j) for i in range(T) for j in range(i, T)], dtype=np.int32
)
P = len(_PAIRS)


def _sc_body(table_hbm, lab_hbm, emb_hbm, e_out_hbm, part_out_hbm,
             lab_v, idxc_v, c_v, e_v, emb_v, part_v,
             counts_sh, zero_v, hlab_v, hcnt_v, ones_v,
             sem_c, sem_e, sem_o, sem_h):
    cid = lax.axis_index("c")
    sid = lax.axis_index("s")
    wid = sid * 2 + cid
    base = wid * ROWS_W

    pltpu.sync_copy(lab_hbm.at[wid], lab_v)
    for k in range(ROWS_W // LANES):
        sl = pl.ds(k * LANES, LANES)
        idxc_v[sl] = lab_v[sl] - 1

    gat_c = pltpu.async_copy(table_hbm.at[idxc_v], c_v, sem_c)
    gat_e = pltpu.async_copy(table_hbm.at[lab_v], e_v, sem_e)

    @pl.when(cid == 0)
    def _():
        def zfill(i, _):
            zero_v[pl.ds(i * LANES, LANES)] = jnp.zeros((LANES,), jnp.float32)
            return 0
        lax.fori_loop(0, ZCH // LANES, zfill, 0)
        for k in range(ROWS_W // LANES):
            ones_v[pl.ds(k * LANES, LANES)] = jnp.ones((LANES,), jnp.float32)
        pltpu.sync_copy(lab_hbm.at[2 * sid + 1], hlab_v)
        pltpu.sync_copy(zero_v, counts_sh.at[pl.ds(sid * ZCH, ZCH)])
        plsc.subcore_barrier()
        pltpu.sync_copy(ones_v, counts_sh.at[lab_v], add=True)
        pltpu.sync_copy(ones_v, counts_sh.at[hlab_v], add=True)
        plsc.subcore_barrier()

    pltpu.sync_copy(emb_hbm.at[pl.ds(base, ROWS_W)], emb_v)

    gat_e.wait()
    put_e = pltpu.async_copy(e_v, e_out_hbm.at[pl.ds(base, ROWS_W)], sem_o)
    gat_c.wait()

    def row_step(r, acc):
        for k in range(K // LANES):
            sl = pl.ds(k * LANES, LANES)
            d = emb_v[r, sl] - c_v[r, sl]
            acc = acc + d * d
        return acc

    acc = lax.fori_loop(0, ROWS_W, row_step, jnp.zeros((LANES,), jnp.float32))

    @pl.when(cid == 0)
    def _():
        hacc = jnp.zeros((LANES,), jnp.float32)
        pltpu.async_copy(counts_sh.at[lab_v], hcnt_v, sem_h).wait()
        for k in range(ROWS_W // LANES):
            hacc = hacc + hcnt_v[pl.ds(k * LANES, LANES)]
        pltpu.async_copy(counts_sh.at[hlab_v], hcnt_v, sem_h).wait()
        for k in range(ROWS_W // LANES):
            hacc = hacc + hcnt_v[pl.ds(k * LANES, LANES)]
        part_v[...] = acc - 0.0625 * hacc

    @pl.when(cid != 0)
    def _():
        part_v[...] = acc

    pltpu.sync_copy(part_v, part_out_hbm.at[wid])
    put_e.wait()


def _sc_gather_center(table, lab2, emb):
    mesh = plsc.VectorSubcoreMesh(core_axis_name="c", subcore_axis_name="s")
    fn = functools.partial(
        pl.kernel,
        out_type=(
            jax.ShapeDtypeStruct((B, K), jnp.float32),
            jax.ShapeDtypeStruct((NW, LANES), jnp.float32),
        ),
        mesh=mesh,
        scratch_types=[
            pltpu.VMEM((ROWS_W,), jnp.int32),
            pltpu.VMEM((ROWS_W,), jnp.int32),
            pltpu.VMEM((ROWS_W, K), jnp.float32),
            pltpu.VMEM((ROWS_W, K), jnp.float32),
            pltpu.VMEM((ROWS_W, K), jnp.float32),
            pltpu.VMEM((LANES,), jnp.float32),
            pltpu.VMEM_SHARED((V_PAD,), jnp.float32),
            pltpu.VMEM((ZCH,), jnp.float32),
            pltpu.VMEM((ROWS_W,), jnp.int32),
            pltpu.VMEM((ROWS_W,), jnp.float32),
            pltpu.VMEM((ROWS_W,), jnp.float32),
            pltpu.SemaphoreType.DMA,
            pltpu.SemaphoreType.DMA,
            pltpu.SemaphoreType.DMA,
            pltpu.SemaphoreType.DMA,
        ],
    )(_sc_body)
    return fn(table, lab2, emb)


def _tc_loss_body(e_ref, part_ref, pairs_ref, out_ref, aa_ref, bb_ref):
    p = pl.program_id(0)

    @pl.when(p == 0)
    def _():
        e = e_ref[...]
        sq = jnp.sum(e * e, axis=1, keepdims=True)
        r = 0.5 - sq
        sqrt2 = np.float32(np.sqrt(2.0))
        s2e = e * sqrt2
        onec = jnp.ones((B, 1), jnp.float32)
        zpad = jnp.zeros((B, AUG - K - 2), jnp.float32)
        aa_ref[...] = jnp.concatenate(
            [s2e, r, onec, zpad], axis=1).astype(jnp.bfloat16)
        bb_ref[...] = jnp.concatenate(
            [s2e, onec, r, zpad], axis=1).astype(jnp.bfloat16)
        out_ref[0, 0] = jnp.sum(part_ref[...])

    @pl.when(p != 0)
    def _():
        ti = pairs_ref[p - 1, 0]
        tj = pairs_ref[p - 1, 1]
        ri = pl.multiple_of(ti * TILE, TILE)
        rj = pl.multiple_of(tj * TILE, TILE)
        ai = aa_ref[pl.ds(ri, TILE), :]
        bj = bb_ref[pl.ds(rj, TILE), :]
        g = lax.dot_general(
            ai, bj, (((1,), (1,)), ((), ())),
            preferred_element_type=jnp.float32,
        )
        hinge = jnp.maximum(0.0, g)
        w = jnp.where(ti == tj, 0.0625, 0.125)
        out_ref[0, 0] += w * jnp.sum(hinge)


def _tc_loss(e_rows, parts, pairs):
    return pl.pallas_call(
        _tc_loss_body,
        grid=(P + 1,),
        in_specs=[
            pl.BlockSpec((B, K), lambda p: (0, 0)),
            pl.BlockSpec((NW, LANES), lambda p: (0, 0)),
            pl.BlockSpec(memory_space=pltpu.SMEM),
        ],
        out_specs=pl.BlockSpec(memory_space=pltpu.SMEM),
        out_shape=jax.ShapeDtypeStruct((1, 1), jnp.float32),
        scratch_shapes=[
            pltpu.VMEM((B, AUG), jnp.bfloat16),
            pltpu.VMEM((B, AUG), jnp.bfloat16),
        ],
    )(e_rows, parts, pairs)


def kernel(embeddings, labels, table):
    labels = labels.astype(jnp.int32)
    lab2 = labels.reshape(NW, ROWS_W)
    e_rows, parts = _sc_gather_center(table, lab2, embeddings)
    pairs = jnp.asarray(_PAIRS)
    loss = _tc_loss(e_rows, parts, pairs)
    return loss[0, 0]


# speedup vs baseline: 1.2485x; 1.2485x over previous
import functools

import jax
import jax.numpy as jnp
import numpy as np
from jax import lax
from jax.experimental import pallas as pl
from jax.experimental.pallas import tpu as pltpu
from jax.experimental.pallas import tpu_sc as plsc

B = 4096
K = 128
NW = 32
ROWS_W = B // NW
LANES = 16

V_PAD = 100352
ZCH = V_PAD // 16
ZSUB = ZCH // 8

AUG = 136
TILE = 1024
T = B // TILE
_PAIRS = np.array(
    [(i, j) for i in range(T) for j in range(i, T)], dtype=np.int32
)
P = len(_PAIRS)


def _sc_body(table_hbm, lab_hbm, emb_hbm, e_out_hbm, part_out_hbm,
             lab_v, idxc_v, c_v, e_v, emb_v, part_v,
             counts_sh, zero_v, hlab_v, hcnt_v, ones_v,
             sem_c, sem_e, sem_o, sem_h):
    cid = lax.axis_index("c")
    sid = lax.axis_index("s")
    wid = sid * 2 + cid
    base = wid * ROWS_W

    pltpu.sync_copy(lab_hbm.at[wid], lab_v)
    for k in range(ROWS_W // LANES):
        sl = pl.ds(k * LANES, LANES)
        idxc_v[sl] = lab_v[sl] - 1

    gat_c = pltpu.async_copy(table_hbm.at[idxc_v], c_v, sem_c)
    gat_e = pltpu.async_copy(table_hbm.at[lab_v], e_v, sem_e)

    @pl.when(cid == 0)
    def _():
        def zfill(i, _):
            zero_v[pl.ds(i * LANES, LANES)] = jnp.zeros((LANES,), jnp.float32)
            return 0
        lax.fori_loop(0, ZSUB // LANES, zfill, 0)
        for k in range(ROWS_W // LANES):
            ones_v[pl.ds(k * LANES, LANES)] = jnp.ones((LANES,), jnp.float32)
        pltpu.sync_copy(lab_hbm.at[2 * sid + 1], hlab_v)
        for j in range(ZCH // ZSUB):
            pltpu.sync_copy(
                zero_v, counts_sh.at[pl.ds(sid * ZCH + j * ZSUB, ZSUB)])
        plsc.subcore_barrier()
        pltpu.sync_copy(ones_v, counts_sh.at[lab_v], add=True)
        pltpu.sync_copy(ones_v, counts_sh.at[hlab_v], add=True)
        plsc.subcore_barrier()

    pltpu.sync_copy(emb_hbm.at[pl.ds(base, ROWS_W)], emb_v)

    gat_e.wait()
    put_e = pltpu.async_copy(e_v, e_out_hbm.at[pl.ds(base, ROWS_W)], sem_o)
    gat_c.wait()

    def row_step(r, acc):
        for k in range(K // LANES):
            sl = pl.ds(k * LANES, LANES)
            d = emb_v[r, sl] - c_v[r, sl]
            acc = acc + d * d
        return acc

    acc = lax.fori_loop(0, ROWS_W, row_step, jnp.zeros((LANES,), jnp.float32))

    @pl.when(cid == 0)
    def _():
        hacc = jnp.zeros((LANES,), jnp.float32)
        pltpu.async_copy(counts_sh.at[lab_v], hcnt_v, sem_h).wait()
        for k in range(ROWS_W // LANES):
            hacc = hacc + hcnt_v[pl.ds(k * LANES, LANES)]
        pltpu.async_copy(counts_sh.at[hlab_v], hcnt_v, sem_h).wait()
        for k in range(ROWS_W // LANES):
            hacc = hacc + hcnt_v[pl.ds(k * LANES, LANES)]
        part_v[...] = acc - 0.0625 * hacc

    @pl.when(cid != 0)
    def _():
        part_v[...] = acc

    pltpu.sync_copy(part_v, part_out_hbm.at[wid])
    put_e.wait()


def _sc_gather_center(table, lab2, emb):
    mesh = plsc.VectorSubcoreMesh(core_axis_name="c", subcore_axis_name="s")
    fn = functools.partial(
        pl.kernel,
        out_type=(
            jax.ShapeDtypeStruct((B, K), jnp.float32),
            jax.ShapeDtypeStruct((NW, LANES), jnp.float32),
        ),
        mesh=mesh,
        scratch_types=[
            pltpu.VMEM((ROWS_W,), jnp.int32),
            pltpu.VMEM((ROWS_W,), jnp.int32),
            pltpu.VMEM((ROWS_W, K), jnp.float32),
            pltpu.VMEM((ROWS_W, K), jnp.float32),
            pltpu.VMEM((ROWS_W, K), jnp.float32),
            pltpu.VMEM((LANES,), jnp.float32),
            pltpu.VMEM_SHARED((V_PAD,), jnp.float32),
            pltpu.VMEM((ZSUB,), jnp.float32),
            pltpu.VMEM((ROWS_W,), jnp.int32),
            pltpu.VMEM((ROWS_W,), jnp.float32),
            pltpu.VMEM((ROWS_W,), jnp.float32),
            pltpu.SemaphoreType.DMA,
            pltpu.SemaphoreType.DMA,
            pltpu.SemaphoreType.DMA,
            pltpu.SemaphoreType.DMA,
        ],
    )(_sc_body)
    return fn(table, lab2, emb)


def _tc_loss_body(e_ref, part_ref, pairs_ref, out_ref, aa_ref, bb_ref,
                  acc_ref):
    p = pl.program_id(0)

    @pl.when(p == 0)
    def _():
        e = e_ref[...]
        sq = jnp.sum(e * e, axis=1, keepdims=True)
        r = 0.5 - sq
        sqrt2 = np.float32(np.sqrt(2.0))
        s2e = e * sqrt2
        onec = jnp.ones((B, 1), jnp.float32)
        zpad = jnp.zeros((B, AUG - K - 2), jnp.float32)
        aa_ref[...] = jnp.concatenate(
            [s2e, r, onec, zpad], axis=1).astype(jnp.bfloat16)
        bb_ref[...] = jnp.concatenate(
            [s2e, onec, r, zpad], axis=1).astype(jnp.bfloat16)
        acc_ref[...] = jnp.zeros((1, TILE), jnp.float32)

    @pl.when(p != 0)
    def _():
        ti = pairs_ref[p - 1, 0]
        tj = pairs_ref[p - 1, 1]
        ri = pl.multiple_of(ti * TILE, TILE)
        rj = pl.multiple_of(tj * TILE, TILE)
        ai = aa_ref[pl.ds(ri, TILE), :]
        bj = bb_ref[pl.ds(rj, TILE), :]
        g = lax.dot_general(
            ai, bj, (((1,), (1,)), ((), ())),
            preferred_element_type=jnp.float32,
        )
        hinge = jnp.maximum(0.0, g)
        w = jnp.where(ti == tj, 0.0625, 0.125)
        acc_ref[...] += w * jnp.sum(hinge, axis=0, keepdims=True)

    @pl.when(p == P)
    def _():
        out_ref[0, 0] = jnp.sum(acc_ref[...]) + jnp.sum(part_ref[...])


def _tc_loss(e_rows, parts, pairs):
    return pl.pallas_call(
        _tc_loss_body,
        grid=(P + 1,),
        in_specs=[
            pl.BlockSpec((B, K), lambda p: (0, 0)),
            pl.BlockSpec((NW, LANES), lambda p: (0, 0)),
            pl.BlockSpec(memory_space=pltpu.SMEM),
        ],
        out_specs=pl.BlockSpec(memory_space=pltpu.SMEM),
        out_shape=jax.ShapeDtypeStruct((1, 1), jnp.float32),
        scratch_shapes=[
            pltpu.VMEM((B, AUG), jnp.bfloat16),
            pltpu.VMEM((B, AUG), jnp.bfloat16),
            pltpu.VMEM((1, TILE), jnp.float32),
        ],
    )(e_rows, parts, pairs)


def kernel(embeddings, labels, table):
    labels = labels.astype(jnp.int32)
    lab2 = labels.reshape(NW, ROWS_W)
    e_rows, parts = _sc_gather_center(table, lab2, embeddings)
    pairs = jnp.asarray(_PAIRS)
    loss = _tc_loss(e_rows, parts, pairs)
    return loss[0, 0]


# speedup vs baseline: 1.2625x; 1.0112x over previous
import functools

import jax
import jax.numpy as jnp
import numpy as np
from jax import lax
from jax.experimental import pallas as pl
from jax.experimental.pallas import tpu as pltpu
from jax.experimental.pallas import tpu_sc as plsc

B = 4096
K = 128
NW = 32
ROWS_W = B // NW
LANES = 16

V_PAD = 100352
ZCH = V_PAD // 16
ZSUB = ZCH // 8

AUG = 136
TILE = 1024
T = B // TILE
_PAIRS = np.array(
    [(i, j) for i in range(T) for j in range(i, T)], dtype=np.int32
)
P = len(_PAIRS)


HALF = ROWS_W // 2


def _sc_body(table_hbm, lab_hbm, emb_hbm, e_out_hbm, part_out_hbm,
             lab_v, idxc_v, c_v, e_v, emb_v, part_v,
             counts_sh, zero_v, hlab_v, hcnt_v, ones_v,
             sem_c, sem_c2, sem_m, sem_e, sem_o, sem_h):
    cid = lax.axis_index("c")
    sid = lax.axis_index("s")
    wid = sid * 2 + cid
    base = wid * ROWS_W

    get_m = pltpu.async_copy(emb_hbm.at[pl.ds(base, ROWS_W)], emb_v, sem_m)
    pltpu.sync_copy(lab_hbm.at[wid], lab_v)
    for k in range(ROWS_W // LANES):
        sl = pl.ds(k * LANES, LANES)
        idxc_v[sl] = lab_v[sl] - 1

    gat_c1 = pltpu.async_copy(
        table_hbm.at[idxc_v.at[pl.ds(0, HALF)]],
        c_v.at[pl.ds(0, HALF)], sem_c)
    gat_c2 = pltpu.async_copy(
        table_hbm.at[idxc_v.at[pl.ds(HALF, HALF)]],
        c_v.at[pl.ds(HALF, HALF)], sem_c2)
    gat_e = pltpu.async_copy(table_hbm.at[lab_v], e_v, sem_e)

    @pl.when(cid == 0)
    def _():
        def zfill(i, _):
            zero_v[pl.ds(i * LANES, LANES)] = jnp.zeros((LANES,), jnp.float32)
            return 0
        lax.fori_loop(0, ZSUB // LANES, zfill, 0)
        for k in range(ROWS_W // LANES):
            ones_v[pl.ds(k * LANES, LANES)] = jnp.ones((LANES,), jnp.float32)
        pltpu.sync_copy(lab_hbm.at[2 * sid + 1], hlab_v)
        for j in range(ZCH // ZSUB):
            pltpu.sync_copy(
                zero_v, counts_sh.at[pl.ds(sid * ZCH + j * ZSUB, ZSUB)])
        plsc.subcore_barrier()
        pltpu.sync_copy(ones_v, counts_sh.at[lab_v], add=True)
        pltpu.sync_copy(ones_v, counts_sh.at[hlab_v], add=True)
        plsc.subcore_barrier()

    def center_half(lo, acc0):
        def row_step(i, a):
            r0 = lo + 2 * i
            for rr in range(2):
                for k in range(K // LANES):
                    sl = pl.ds(k * LANES, LANES)
                    d = emb_v[r0 + rr, sl] - c_v[r0 + rr, sl]
                    a = a + d * d
            return a
        return lax.fori_loop(0, HALF // 2, row_step, acc0)

    gat_e.wait()
    put_e = pltpu.async_copy(e_v, e_out_hbm.at[pl.ds(base, ROWS_W)], sem_o)
    get_m.wait()
    gat_c1.wait()
    acc = center_half(0, jnp.zeros((LANES,), jnp.float32))
    gat_c2.wait()
    acc = center_half(HALF, acc)

    @pl.when(cid == 0)
    def _():
        hacc = jnp.zeros((LANES,), jnp.float32)
        pltpu.async_copy(counts_sh.at[lab_v], hcnt_v, sem_h).wait()
        for k in range(ROWS_W // LANES):
            hacc = hacc + hcnt_v[pl.ds(k * LANES, LANES)]
        pltpu.async_copy(counts_sh.at[hlab_v], hcnt_v, sem_h).wait()
        for k in range(ROWS_W // LANES):
            hacc = hacc + hcnt_v[pl.ds(k * LANES, LANES)]
        part_v[...] = acc - 0.0625 * hacc

    @pl.when(cid != 0)
    def _():
        part_v[...] = acc

    pltpu.sync_copy(part_v, part_out_hbm.at[wid])
    put_e.wait()


def _sc_gather_center(table, lab2, emb):
    mesh = plsc.VectorSubcoreMesh(core_axis_name="c", subcore_axis_name="s")
    fn = functools.partial(
        pl.kernel,
        out_type=(
            jax.ShapeDtypeStruct((B, K), jnp.float32),
            jax.ShapeDtypeStruct((NW, LANES), jnp.float32),
        ),
        mesh=mesh,
        scratch_types=[
            pltpu.VMEM((ROWS_W,), jnp.int32),
            pltpu.VMEM((ROWS_W,), jnp.int32),
            pltpu.VMEM((ROWS_W, K), jnp.float32),
            pltpu.VMEM((ROWS_W, K), jnp.float32),
            pltpu.VMEM((ROWS_W, K), jnp.float32),
            pltpu.VMEM((LANES,), jnp.float32),
            pltpu.VMEM_SHARED((V_PAD,), jnp.float32),
            pltpu.VMEM((ZSUB,), jnp.float32),
            pltpu.VMEM((ROWS_W,), jnp.int32),
            pltpu.VMEM((ROWS_W,), jnp.float32),
            pltpu.VMEM((ROWS_W,), jnp.float32),
            pltpu.SemaphoreType.DMA,
            pltpu.SemaphoreType.DMA,
            pltpu.SemaphoreType.DMA,
            pltpu.SemaphoreType.DMA,
            pltpu.SemaphoreType.DMA,
            pltpu.SemaphoreType.DMA,
        ],
    )(_sc_body)
    return fn(table, lab2, emb)


def _tc_loss_body(e_ref, part_ref, pairs_ref, out_ref, aa_ref, bb_ref,
                  acc_ref):
    p = pl.program_id(0)

    @pl.when(p == 0)
    def _():
        e = e_ref[...]
        sq = jnp.sum(e * e, axis=1, keepdims=True)
        r = 0.5 - sq
        sqrt2 = np.float32(np.sqrt(2.0))
        s2e = e * sqrt2
        onec = jnp.ones((B, 1), jnp.float32)
        zpad = jnp.zeros((B, AUG - K - 2), jnp.float32)
        aa_ref[...] = jnp.concatenate(
            [s2e, r, onec, zpad], axis=1).astype(jnp.bfloat16)
        bb_ref[...] = jnp.concatenate(
            [s2e, onec, r, zpad], axis=1).astype(jnp.bfloat16)
        acc_ref[...] = jnp.zeros((1, TILE), jnp.float32)

    @pl.when(p != 0)
    def _():
        ti = pairs_ref[p - 1, 0]
        tj = pairs_ref[p - 1, 1]
        ri = pl.multiple_of(ti * TILE, TILE)
        w = jnp.where(ti == tj, 0.0625, 0.125)
        ai = aa_ref[pl.ds(ri, TILE), :]
        for hh in range(2):
            rj = pl.multiple_of(tj * TILE + hh * (TILE // 2), TILE // 2)
            bj = bb_ref[pl.ds(rj, TILE // 2), :]
            g = lax.dot_general(
                ai, bj, (((1,), (1,)), ((), ())),
                preferred_element_type=jnp.float32,
            )
            hinge = jnp.maximum(0.0, g)
            sl = pl.ds(hh * (TILE // 2), TILE // 2)
            acc_ref[:, sl] += w * jnp.sum(hinge, axis=0, keepdims=True)

    @pl.when(p == P)
    def _():
        out_ref[0, 0] = jnp.sum(acc_ref[...]) + jnp.sum(part_ref[...])


def _tc_loss(e_rows, parts, pairs):
    return pl.pallas_call(
        _tc_loss_body,
        grid=(P + 1,),
        in_specs=[
            pl.BlockSpec((B, K), lambda p: (0, 0)),
            pl.BlockSpec((NW, LANES), lambda p: (0, 0)),
            pl.BlockSpec(memory_space=pltpu.SMEM),
        ],
        out_specs=pl.BlockSpec(memory_space=pltpu.SMEM),
        out_shape=jax.ShapeDtypeStruct((1, 1), jnp.float32),
        scratch_shapes=[
            pltpu.VMEM((B, AUG), jnp.bfloat16),
            pltpu.VMEM((B, AUG), jnp.bfloat16),
            pltpu.VMEM((1, TILE), jnp.float32),
        ],
    )(e_rows, parts, pairs)


def kernel(embeddings, labels, table):
    labels = labels.astype(jnp.int32)
    lab2 = labels.reshape(NW, ROWS_W)
    e_rows, parts = _sc_gather_center(table, lab2, embeddings)
    pairs = jnp.asarray(_PAIRS)
    loss = _tc_loss(e_rows, parts, pairs)
    return loss[0, 0]


# speedup vs baseline: 1.3663x; 1.0822x over previous
import functools

import jax
import jax.numpy as jnp
import numpy as np
from jax import lax
from jax.experimental import pallas as pl
from jax.experimental.pallas import tpu as pltpu
from jax.experimental.pallas import tpu_sc as plsc

B = 4096
K = 128
NW = 32
ROWS_W = B // NW
LANES = 16

V_PAD = 100352
ZCH = V_PAD // 16
ZSUB = ZCH // 8

AUG = 136
TILE = 1024
T = B // TILE
_PAIRS = np.array(
    [(i, j) for i in range(T) for j in range(i, T)], dtype=np.int32
)
P = len(_PAIRS)


HALF = ROWS_W // 2


def _sc_body(table_hbm, lab_hbm, emb_hbm, e_out_hbm, part_out_hbm,
             lab_v, idxc_v, c_v, e_v, emb_v, part_v,
             counts_sh, zero_v, hlab_v, hcnt_v, ones_v,
             sem_c, sem_c2, sem_m, sem_e, sem_o, sem_h):
    cid = lax.axis_index("c")
    sid = lax.axis_index("s")
    wid = sid * 2 + cid
    base = wid * ROWS_W

    get_m = pltpu.async_copy(emb_hbm.at[pl.ds(base, ROWS_W)], emb_v, sem_m)
    pltpu.sync_copy(lab_hbm.at[wid], lab_v)
    for k in range(ROWS_W // LANES):
        sl = pl.ds(k * LANES, LANES)
        idxc_v[sl] = lab_v[sl] - 1

    gat_c1 = pltpu.async_copy(
        table_hbm.at[idxc_v.at[pl.ds(0, HALF)]],
        c_v.at[pl.ds(0, HALF)], sem_c)
    gat_c2 = pltpu.async_copy(
        table_hbm.at[idxc_v.at[pl.ds(HALF, HALF)]],
        c_v.at[pl.ds(HALF, HALF)], sem_c2)
    gat_e = pltpu.async_copy(table_hbm.at[lab_v], e_v, sem_e)

    @pl.when(cid == 0)
    def _():
        def zfill(i, _):
            zero_v[pl.ds(i * LANES, LANES)] = jnp.zeros((LANES,), jnp.float32)
            return 0
        lax.fori_loop(0, ZSUB // LANES, zfill, 0)
        for k in range(ROWS_W // LANES):
            ones_v[pl.ds(k * LANES, LANES)] = jnp.ones((LANES,), jnp.float32)
        pltpu.sync_copy(lab_hbm.at[2 * sid + 1], hlab_v)
        for j in range(ZCH // ZSUB):
            pltpu.sync_copy(
                zero_v, counts_sh.at[pl.ds(sid * ZCH + j * ZSUB, ZSUB)])
        plsc.subcore_barrier()
        pltpu.sync_copy(ones_v, counts_sh.at[lab_v], add=True)
        pltpu.sync_copy(ones_v, counts_sh.at[hlab_v], add=True)
        plsc.subcore_barrier()

    def center_half(lo, acc0):
        def row_step(i, a):
            r0 = lo + 2 * i
            for rr in range(2):
                for k in range(K // LANES):
                    sl = pl.ds(k * LANES, LANES)
                    d = emb_v[r0 + rr, sl] - c_v[r0 + rr, sl]
                    a = a + d * d
            return a
        return lax.fori_loop(0, HALF // 2, row_step, acc0)

    gat_e.wait()
    put_e = pltpu.async_copy(e_v, e_out_hbm.at[pl.ds(base, ROWS_W)], sem_o)
    get_m.wait()
    gat_c1.wait()
    acc = center_half(0, jnp.zeros((LANES,), jnp.float32))
    gat_c2.wait()
    acc = center_half(HALF, acc)

    @pl.when(cid == 0)
    def _():
        hacc = jnp.zeros((LANES,), jnp.float32)
        pltpu.async_copy(counts_sh.at[lab_v], hcnt_v, sem_h).wait()
        for k in range(ROWS_W // LANES):
            hacc = hacc + hcnt_v[pl.ds(k * LANES, LANES)]
        pltpu.async_copy(counts_sh.at[hlab_v], hcnt_v, sem_h).wait()
        for k in range(ROWS_W // LANES):
            hacc = hacc + hcnt_v[pl.ds(k * LANES, LANES)]
        part_v[...] = acc - 0.0625 * hacc

    @pl.when(cid != 0)
    def _():
        part_v[...] = acc

    pltpu.sync_copy(part_v, part_out_hbm.at[wid])
    put_e.wait()


def _sc_gather_center(table, lab2, emb):
    mesh = plsc.VectorSubcoreMesh(core_axis_name="c", subcore_axis_name="s")
    fn = functools.partial(
        pl.kernel,
        out_type=(
            jax.ShapeDtypeStruct((B, K), jnp.float32),
            jax.ShapeDtypeStruct((NW, LANES), jnp.float32),
        ),
        mesh=mesh,
        scratch_types=[
            pltpu.VMEM((ROWS_W,), jnp.int32),
            pltpu.VMEM((ROWS_W,), jnp.int32),
            pltpu.VMEM((ROWS_W, K), jnp.float32),
            pltpu.VMEM((ROWS_W, K), jnp.float32),
            pltpu.VMEM((ROWS_W, K), jnp.float32),
            pltpu.VMEM((LANES,), jnp.float32),
            pltpu.VMEM_SHARED((V_PAD,), jnp.float32),
            pltpu.VMEM((ZSUB,), jnp.float32),
            pltpu.VMEM((ROWS_W,), jnp.int32),
            pltpu.VMEM((ROWS_W,), jnp.float32),
            pltpu.VMEM((ROWS_W,), jnp.float32),
            pltpu.SemaphoreType.DMA,
            pltpu.SemaphoreType.DMA,
            pltpu.SemaphoreType.DMA,
            pltpu.SemaphoreType.DMA,
            pltpu.SemaphoreType.DMA,
            pltpu.SemaphoreType.DMA,
        ],
    )(_sc_body)
    return fn(table, lab2, emb)


def _tc_loss_body(e_ref, part_ref, out_ref, aa_ref, bb_ref):
    p = pl.program_id(0)

    @pl.when(p == 0)
    def _():
        e = e_ref[...]
        sq = jnp.sum(e * e, axis=1, keepdims=True)
        r = 0.5 - sq
        sqrt2 = np.float32(np.sqrt(2.0))
        s2e = e * sqrt2
        onec = jnp.ones((B, 1), jnp.float32)
        zpad = jnp.zeros((B, AUG - K - 2), jnp.float32)
        aa_ref[...] = jnp.concatenate(
            [s2e, r, onec, zpad], axis=1).astype(jnp.bfloat16)
        bb_ref[...] = jnp.concatenate(
            [s2e, onec, r, zpad], axis=1).astype(jnp.bfloat16)

    @pl.when(p == 1)
    def _():
        sums = []
        for ti, tj in _PAIRS:
            w = np.float32(0.0625 if ti == tj else 0.125)
            ai = aa_ref[ti * TILE:(ti + 1) * TILE, :]
            for hh in range(2):
                c0 = tj * TILE + hh * (TILE // 2)
                bj = bb_ref[c0:c0 + TILE // 2, :]
                g = lax.dot_general(
                    ai, bj, (((1,), (1,)), ((), ())),
                    preferred_element_type=jnp.float32,
                )
                hinge = jnp.maximum(0.0, g)
                sums.append(w * jnp.sum(hinge, axis=0, keepdims=True))
        tot = sums[0]
        for s in sums[1:]:
            tot = tot + s
        out_ref[0, 0] = jnp.sum(tot) + jnp.sum(part_ref[...])


def _tc_loss(e_rows, parts):
    return pl.pallas_call(
        _tc_loss_body,
        grid=(2,),
        in_specs=[
            pl.BlockSpec((B, K), lambda p: (0, 0)),
            pl.BlockSpec((NW, LANES), lambda p: (0, 0)),
        ],
        out_specs=pl.BlockSpec(memory_space=pltpu.SMEM),
        out_shape=jax.ShapeDtypeStruct((1, 1), jnp.float32),
        scratch_shapes=[
            pltpu.VMEM((B, AUG), jnp.bfloat16),
            pltpu.VMEM((B, AUG), jnp.bfloat16),
        ],
    )(e_rows, parts)


def kernel(embeddings, labels, table):
    labels = labels.astype(jnp.int32)
    lab2 = labels.reshape(NW, ROWS_W)
    e_rows, parts = _sc_gather_center(table, lab2, embeddings)
    loss = _tc_loss(e_rows, parts)
    return loss[0, 0]


# speedup vs baseline: 1.4048x; 1.0282x over previous
import functools

import jax
import jax.numpy as jnp
import numpy as np
from jax import lax
from jax.experimental import pallas as pl
from jax.experimental.pallas import tpu as pltpu
from jax.experimental.pallas import tpu_sc as plsc

B = 4096
K = 128
NW = 32
ROWS_W = B // NW
LANES = 16

V_PAD = 100352
ZCH = V_PAD // 16
ZSUB = ZCH // 8

AUG = 136
TILE = 512
T = B // TILE
_PAIRS = np.array(
    [(i, j) for i in range(T) for j in range(i, T)], dtype=np.int32
)
P = len(_PAIRS)


HALF = ROWS_W // 2


def _sc_body(table_hbm, lab_hbm, emb_hbm, e_out_hbm, part_out_hbm,
             lab_v, idxc_v, c_v, e_v, emb_v, part_v,
             counts_sh, zero_v, hlab_v, hcnt_v, ones_v,
             sem_c, sem_c2, sem_m, sem_e, sem_o, sem_h):
    cid = lax.axis_index("c")
    sid = lax.axis_index("s")
    wid = sid * 2 + cid
    base = wid * ROWS_W

    get_m = pltpu.async_copy(emb_hbm.at[pl.ds(base, ROWS_W)], emb_v, sem_m)
    pltpu.sync_copy(lab_hbm.at[wid], lab_v)
    for k in range(ROWS_W // LANES):
        sl = pl.ds(k * LANES, LANES)
        idxc_v[sl] = lab_v[sl] - 1

    gat_c1 = pltpu.async_copy(
        table_hbm.at[idxc_v.at[pl.ds(0, HALF)]],
        c_v.at[pl.ds(0, HALF)], sem_c)
    gat_c2 = pltpu.async_copy(
        table_hbm.at[idxc_v.at[pl.ds(HALF, HALF)]],
        c_v.at[pl.ds(HALF, HALF)], sem_c2)
    gat_e = pltpu.async_copy(table_hbm.at[lab_v], e_v, sem_e)

    @pl.when(cid == 0)
    def _():
        def zfill(i, _):
            zero_v[pl.ds(i * LANES, LANES)] = jnp.zeros((LANES,), jnp.float32)
            return 0
        lax.fori_loop(0, ZSUB // LANES, zfill, 0)
        for k in range(ROWS_W // LANES):
            ones_v[pl.ds(k * LANES, LANES)] = jnp.ones((LANES,), jnp.float32)
        pltpu.sync_copy(lab_hbm.at[2 * sid + 1], hlab_v)
        for j in range(ZCH // ZSUB):
            pltpu.sync_copy(
                zero_v, counts_sh.at[pl.ds(sid * ZCH + j * ZSUB, ZSUB)])
        plsc.subcore_barrier()
        pltpu.sync_copy(ones_v, counts_sh.at[lab_v], add=True)
        pltpu.sync_copy(ones_v, counts_sh.at[hlab_v], add=True)
        plsc.subcore_barrier()

    def center_half(lo, acc0):
        def row_step(i, a):
            r0 = lo + 2 * i
            for rr in range(2):
                for k in range(K // LANES):
                    sl = pl.ds(k * LANES, LANES)
                    d = emb_v[r0 + rr, sl] - c_v[r0 + rr, sl]
                    a = a + d * d
            return a
        return lax.fori_loop(0, HALF // 2, row_step, acc0)

    gat_e.wait()
    put_e = pltpu.async_copy(e_v, e_out_hbm.at[pl.ds(base, ROWS_W)], sem_o)
    get_m.wait()
    gat_c1.wait()
    acc = center_half(0, jnp.zeros((LANES,), jnp.float32))
    gat_c2.wait()
    acc = center_half(HALF, acc)

    @pl.when(cid == 0)
    def _():
        hacc = jnp.zeros((LANES,), jnp.float32)
        pltpu.async_copy(counts_sh.at[lab_v], hcnt_v, sem_h).wait()
        for k in range(ROWS_W // LANES):
            hacc = hacc + hcnt_v[pl.ds(k * LANES, LANES)]
        pltpu.async_copy(counts_sh.at[hlab_v], hcnt_v, sem_h).wait()
        for k in range(ROWS_W // LANES):
            hacc = hacc + hcnt_v[pl.ds(k * LANES, LANES)]
        part_v[...] = acc - 0.0625 * hacc

    @pl.when(cid != 0)
    def _():
        part_v[...] = acc

    pltpu.sync_copy(part_v, part_out_hbm.at[wid])
    put_e.wait()


def _sc_gather_center(table, lab2, emb):
    mesh = plsc.VectorSubcoreMesh(core_axis_name="c", subcore_axis_name="s")
    fn = functools.partial(
        pl.kernel,
        out_type=(
            jax.ShapeDtypeStruct((B, K), jnp.float32),
            jax.ShapeDtypeStruct((NW, LANES), jnp.float32),
        ),
        mesh=mesh,
        scratch_types=[
            pltpu.VMEM((ROWS_W,), jnp.int32),
            pltpu.VMEM((ROWS_W,), jnp.int32),
            pltpu.VMEM((ROWS_W, K), jnp.float32),
            pltpu.VMEM((ROWS_W, K), jnp.float32),
            pltpu.VMEM((ROWS_W, K), jnp.float32),
            pltpu.VMEM((LANES,), jnp.float32),
            pltpu.VMEM_SHARED((V_PAD,), jnp.float32),
            pltpu.VMEM((ZSUB,), jnp.float32),
            pltpu.VMEM((ROWS_W,), jnp.int32),
            pltpu.VMEM((ROWS_W,), jnp.float32),
            pltpu.VMEM((ROWS_W,), jnp.float32),
            pltpu.SemaphoreType.DMA,
            pltpu.SemaphoreType.DMA,
            pltpu.SemaphoreType.DMA,
            pltpu.SemaphoreType.DMA,
            pltpu.SemaphoreType.DMA,
            pltpu.SemaphoreType.DMA,
        ],
    )(_sc_body)
    return fn(table, lab2, emb)


def _tc_loss_body(e_ref, part_ref, out_ref, aa_ref, bb_ref):
    p = pl.program_id(0)

    @pl.when(p == 0)
    def _():
        e = e_ref[...]
        sq = jnp.sum(e * e, axis=1, keepdims=True)
        r = 0.5 - sq
        sqrt2 = np.float32(np.sqrt(2.0))
        s2e = e * sqrt2
        onec = jnp.ones((B, 1), jnp.float32)
        zpad = jnp.zeros((B, AUG - K - 2), jnp.float32)
        aa_ref[...] = jnp.concatenate(
            [s2e, r, onec, zpad], axis=1).astype(jnp.bfloat16)
        bb_ref[...] = jnp.concatenate(
            [s2e, onec, r, zpad], axis=1).astype(jnp.bfloat16)

    @pl.when(p == 1)
    def _():
        sums = []
        for ti, tj in _PAIRS:
            w = np.float32(0.0625 if ti == tj else 0.125)
            ai = aa_ref[ti * TILE:(ti + 1) * TILE, :]
            bj = bb_ref[tj * TILE:(tj + 1) * TILE, :]
            g = lax.dot_general(
                ai, bj, (((1,), (1,)), ((), ())),
                preferred_element_type=jnp.float32,
            )
            hinge = jnp.maximum(0.0, g)
            sums.append(w * jnp.sum(hinge, axis=0, keepdims=True))
        tot = sums[0]
        for s in sums[1:]:
            tot = tot + s
        out_ref[0, 0] = jnp.sum(tot) + jnp.sum(part_ref[...])


def _tc_loss(e_rows, parts):
    return pl.pallas_call(
        _tc_loss_body,
        grid=(2,),
        in_specs=[
            pl.BlockSpec((B, K), lambda p: (0, 0)),
            pl.BlockSpec((NW, LANES), lambda p: (0, 0)),
        ],
        out_specs=pl.BlockSpec(memory_space=pltpu.SMEM),
        out_shape=jax.ShapeDtypeStruct((1, 1), jnp.float32),
        scratch_shapes=[
            pltpu.VMEM((B, AUG), jnp.bfloat16),
            pltpu.VMEM((B, AUG), jnp.bfloat16),
        ],
    )(e_rows, parts)


def kernel(embeddings, labels, table):
    labels = labels.astype(jnp.int32)
    lab2 = labels.reshape(NW, ROWS_W)
    e_rows, parts = _sc_gather_center(table, lab2, embeddings)
    loss = _tc_loss(e_rows, parts)
    return loss[0, 0]
